# baseline (speedup 1.0000x reference)
import functools

import jax
import jax.numpy as jnp
from jax import lax
from jax.experimental import pallas as pl
from jax.experimental.pallas import tpu as pltpu
from jax.experimental.pallas import tpu_sc as plsc

D_MODEL = 64

_info = plsc.get_sparse_core_info()
_NC, _NS = _info.num_cores, _info.num_subcores
_NW = _NC * _NS


@functools.partial(jax.jit, static_argnames=("B", "C"))
def _gather_rows(idx, table, *, B, C):
    b_per_w = B // _NW
    n_chunks = b_per_w // C
    mesh = plsc.VectorSubcoreMesh(core_axis_name="c", subcore_axis_name="s")

    @functools.partial(
        pl.kernel,
        mesh=mesh,
        out_type=jax.ShapeDtypeStruct((B, D_MODEL), jnp.float32),
        scratch_types=[
            pltpu.VMEM((C,), jnp.int32),
            pltpu.VMEM((C, D_MODEL), jnp.float32),
            pltpu.SemaphoreType.DMA,
        ],
        compiler_params=pltpu.CompilerParams(use_tc_tiling_on_sc=False),
    )
    def k(table_hbm, idx_hbm, out_hbm, idx_v, rows_v, sem):
        wid = lax.axis_index("s") * _NC + lax.axis_index("c")
        base = wid * b_per_w

        def body(g, carry):
            off = base + g * C
            pltpu.sync_copy(idx_hbm.at[pl.ds(off, C)], idx_v)
            pltpu.async_copy(table_hbm.at[idx_v], rows_v, sem).wait()
            pltpu.sync_copy(rows_v, out_hbm.at[pl.ds(off, C)])
            return carry

        lax.fori_loop(0, n_chunks, body, 0)

    return k(table, idx)


def kernel(x, table):
    orig_shape = x.shape
    idx = x.reshape(-1).astype(jnp.int32)
    out = _gather_rows(idx, table, B=idx.shape[0], C=800)
    return out.reshape(*orig_shape, D_MODEL)


# speedup vs baseline: 1.0215x; 1.0215x over previous
import functools

import jax
import jax.numpy as jnp
from jax import lax
from jax.experimental import pallas as pl
from jax.experimental.pallas import tpu as pltpu
from jax.experimental.pallas import tpu_sc as plsc

D_MODEL = 64
NBUF = 4

_info = plsc.get_sparse_core_info()
_NC, _NS = _info.num_cores, _info.num_subcores
_NW = _NC * _NS


@functools.partial(jax.jit, static_argnames=("B", "C"))
def _gather_rows(idx, table, *, B, C):
    b_per_w = B // _NW
    n_chunks = b_per_w // C
    assert n_chunks % NBUF == 0 and n_chunks >= 2 * NBUF
    mesh = plsc.VectorSubcoreMesh(core_axis_name="c", subcore_axis_name="s")

    @functools.partial(
        pl.kernel,
        mesh=mesh,
        out_type=jax.ShapeDtypeStruct((B, D_MODEL), jnp.float32),
        scratch_types=[
            pltpu.VMEM((b_per_w,), jnp.int32),
            *([pltpu.VMEM((C, D_MODEL), jnp.float32)] * NBUF),
            *([pltpu.SemaphoreType.DMA] * (2 * NBUF)),
        ],
        compiler_params=pltpu.CompilerParams(use_tc_tiling_on_sc=False),
    )
    def k(table_hbm, idx_hbm, out_hbm, idx_v, *bufs_and_sems):
        bufs = bufs_and_sems[:NBUF]
        gsems = bufs_and_sems[NBUF:2 * NBUF]
        osems = bufs_and_sems[2 * NBUF:]
        wid = lax.axis_index("s") * _NC + lax.axis_index("c")
        base = wid * b_per_w
        pltpu.sync_copy(idx_hbm.at[pl.ds(base, b_per_w)], idx_v)

        def gstart(c, b):
            pltpu.make_async_copy(
                table_hbm.at[idx_v.at[pl.ds(c * C, C)]], bufs[b], gsems[b]
            ).start()

        def gwait(b):
            pltpu.make_async_copy(
                table_hbm.at[idx_v.at[pl.ds(0, C)]], bufs[b], gsems[b]
            ).wait()

        def ostart(c, b):
            pltpu.make_async_copy(
                bufs[b], out_hbm.at[pl.ds(base + c * C, C)], osems[b]
            ).start()

        def owait(b):
            pltpu.make_async_copy(
                bufs[b], out_hbm.at[pl.ds(base, C)], osems[b]
            ).wait()

        gstart(0, 0)
        gstart(1, 1)

        def body(g, carry):
            t0 = g * NBUF
            for b in range(NBUF):
                c = t0 + b
                nb = (b + 2) % NBUF

                @pl.when(jnp.logical_and(c >= 2, c + 2 < n_chunks))
                def _():
                    owait(nb)

                @pl.when(c + 2 < n_chunks)
                def _():
                    gstart(c + 2, nb)

                gwait(b)
                ostart(c, b)
            return carry

        lax.fori_loop(0, n_chunks // NBUF, body, 0)
        for b in range(NBUF):
            owait(b)

    return k(table, idx)


def kernel(x, table):
    orig_shape = x.shape
    idx = x.reshape(-1).astype(jnp.int32)
    out = _gather_rows(idx, table, B=idx.shape[0], C=400)
    return out.reshape(*orig_shape, D_MODEL)


# speedup vs baseline: 1.3565x; 1.3279x over previous
import functools

import jax
import jax.numpy as jnp
from jax import lax
from jax.experimental import pallas as pl
from jax.experimental.pallas import tpu as pltpu
from jax.experimental.pallas import tpu_sc as plsc

D_MODEL = 64
NBUF = 4

_info = plsc.get_sparse_core_info()
_NC, _NS = _info.num_cores, _info.num_subcores
_NW = _NC * _NS


@functools.partial(jax.jit, static_argnames=("B", "C"))
def _gather_rows(idx, table_pad, *, B, C):
    b_per_w = B // _NW
    n_chunks = b_per_w // C
    assert n_chunks % NBUF == 0 and n_chunks >= 2 * NBUF
    assert C % 4 == 0
    mesh = plsc.VectorSubcoreMesh(core_axis_name="c", subcore_axis_name="s")

    @functools.partial(
        pl.kernel,
        mesh=mesh,
        out_type=jax.ShapeDtypeStruct((B, 128), jnp.float32),
        scratch_types=[
            pltpu.VMEM((b_per_w,), jnp.int32),
            *([pltpu.VMEM((C, 128), jnp.float32)] * NBUF),
            *([pltpu.SemaphoreType.DMA] * (2 * NBUF)),
        ],
        compiler_params=pltpu.CompilerParams(use_tc_tiling_on_sc=False),
    )
    def k(table_hbm, idx_hbm, out_hbm, idx_v, *bufs_and_sems):
        bufs = bufs_and_sems[:NBUF]
        gsems = bufs_and_sems[NBUF:2 * NBUF]
        osems = bufs_and_sems[2 * NBUF:]
        wid = lax.axis_index("s") * _NC + lax.axis_index("c")
        base = wid * b_per_w
        pltpu.sync_copy(idx_hbm.at[pl.ds(base, b_per_w)], idx_v)

        def gstart(c, b):
            pltpu.make_async_copy(
                table_hbm.at[idx_v.at[pl.ds(c * C, C)]],
                bufs[b],
                gsems[b],
            ).start()

        def gwait(b):
            pltpu.make_async_copy(
                table_hbm.at[idx_v.at[pl.ds(0, C)]],
                bufs[b],
                gsems[b],
            ).wait()

        def ostart(c, b):
            pltpu.make_async_copy(
                bufs[b].at[pl.ds(0, C), pl.ds(0, D_MODEL)],
                out_hbm.at[pl.ds(base + c * C, C), pl.ds(0, D_MODEL)],
                osems[b],
            ).start()

        def owait(b):
            pltpu.make_async_copy(
                bufs[b].at[pl.ds(0, C), pl.ds(0, D_MODEL)],
                out_hbm.at[pl.ds(base, C), pl.ds(0, D_MODEL)],
                osems[b],
            ).wait()

        gstart(0, 0)
        gstart(1, 1)

        def body(g, carry):
            t0 = g * NBUF
            for b in range(NBUF):
                c = t0 + b
                nb = (b + 2) % NBUF

                @pl.when(jnp.logical_and(c >= 2, c + 2 < n_chunks))
                def _():
                    owait(nb)

                @pl.when(c + 2 < n_chunks)
                def _():
                    gstart(c + 2, nb)

                gwait(b)
                ostart(c, b)
            return carry

        lax.fori_loop(0, n_chunks // NBUF, body, 0)
        for b in range(NBUF):
            owait(b)

    return k(table_pad, idx)


def kernel(x, table):
    orig_shape = x.shape
    idx = x.reshape(-1).astype(jnp.int32)
    table_pad = jnp.pad(table, ((0, 0), (0, 128 - D_MODEL)))
    out_pad = _gather_rows(idx, table_pad, B=idx.shape[0], C=200)
    return out_pad[:, :D_MODEL].reshape(*orig_shape, D_MODEL)
